# baseline (speedup 1.0000x reference)
import jax
import jax.numpy as jnp
from jax.experimental import pallas as pl

_ANCH_W = (18.7807, 28.8912, 48.6849)
_ANCH_H = (33.4659, 61.7536, 68.3897)
_STRIDE = 8.0


def _decode_body(bbox_ref, conf_ref, cls_ref, bbox_out, idx_out, score_out):
    a = pl.program_id(0) % 3

    lane = jax.lax.broadcasted_iota(jnp.int32, (128, 640), 1)
    row = jax.lax.broadcasted_iota(jnp.int32, (128, 640), 0)
    c = lane % 5
    xf = (lane // 5).astype(jnp.float32)
    yf = row.astype(jnp.float32)

    aw = jnp.where(a == 0, _ANCH_W[0], jnp.where(a == 1, _ANCH_W[1], _ANCH_W[2]))
    ah = jnp.where(a == 0, _ANCH_H[0], jnp.where(a == 1, _ANCH_H[1], _ANCH_H[2]))

    v = bbox_ref[0]
    sig = jax.nn.sigmoid(v)
    ex = jnp.exp(v)

    out = jnp.where(
        c == 0,
        (sig + xf) * _STRIDE,
        jnp.where(
            c == 1,
            (sig + yf) * _STRIDE,
            jnp.where(c == 2, ex * aw, jnp.where(c == 3, ex * ah, sig * 360.0 - 180.0)),
        ),
    )
    bbox_out[0] = out

    score_out[0] = jax.nn.sigmoid(conf_ref[0]) * jax.nn.sigmoid(cls_ref[0])
    idx_out[0] = jnp.zeros((128, 128), jnp.int32)


@jax.jit
def kernel(bbox, conf, cls_logits):
    nB, nA, nH, nW, _ = bbox.shape
    g = nB * nA
    bbox_r = bbox.reshape(g, nH, nW * 5)
    conf_r = conf.reshape(g, nH, nW)
    cls_r = cls_logits.reshape(g, nH, nW)

    bbox_o, idx_o, score_o = pl.pallas_call(
        _decode_body,
        grid=(g,),
        in_specs=[
            pl.BlockSpec((1, nH, nW * 5), lambda i: (i, 0, 0)),
            pl.BlockSpec((1, nH, nW), lambda i: (i, 0, 0)),
            pl.BlockSpec((1, nH, nW), lambda i: (i, 0, 0)),
        ],
        out_specs=[
            pl.BlockSpec((1, nH, nW * 5), lambda i: (i, 0, 0)),
            pl.BlockSpec((1, nH, nW), lambda i: (i, 0, 0)),
            pl.BlockSpec((1, nH, nW), lambda i: (i, 0, 0)),
        ],
        out_shape=[
            jax.ShapeDtypeStruct((g, nH, nW * 5), jnp.float32),
            jax.ShapeDtypeStruct((g, nH, nW), jnp.int32),
            jax.ShapeDtypeStruct((g, nH, nW), jnp.float32),
        ],
    )(bbox_r, conf_r, cls_r)

    return (
        bbox_o.reshape(nB, nA * nH * nW, 5),
        idx_o.reshape(nB, nA * nH * nW),
        score_o.reshape(nB, nA * nH * nW),
    )


# speedup vs baseline: 10.2250x; 10.2250x over previous
import jax
import jax.numpy as jnp
from jax.experimental import pallas as pl

_ANCH_W = (18.7807, 28.8912, 48.6849)
_ANCH_H = (33.4659, 61.7536, 68.3897)
_STRIDE = 8.0


def _decode_body(bbox_ref, conf_ref, cls_ref, bbox_out, idx_out, score_out):
    a = pl.program_id(0) % 3
    aw = jnp.where(a == 0, _ANCH_W[0], jnp.where(a == 1, _ANCH_W[1], _ANCH_W[2]))
    ah = jnp.where(a == 0, _ANCH_H[0], jnp.where(a == 1, _ANCH_H[1], _ANCH_H[2]))

    xf = jax.lax.broadcasted_iota(jnp.int32, (128, 128), 1).astype(jnp.float32)
    yf = jax.lax.broadcasted_iota(jnp.int32, (128, 128), 0).astype(jnp.float32)

    tx = bbox_ref[0, 0]
    ty = bbox_ref[0, 1]
    tw = bbox_ref[0, 2]
    th = bbox_ref[0, 3]
    tr = bbox_ref[0, 4]

    bbox_out[0, 0] = (jax.nn.sigmoid(tx) + xf) * _STRIDE
    bbox_out[1, 0] = (jax.nn.sigmoid(ty) + yf) * _STRIDE
    bbox_out[2, 0] = jnp.exp(tw) * aw
    bbox_out[3, 0] = jnp.exp(th) * ah
    bbox_out[4, 0] = jax.nn.sigmoid(tr) * 360.0 - 180.0

    score_out[0] = jax.nn.sigmoid(conf_ref[0]) * jax.nn.sigmoid(cls_ref[0])
    idx_out[0] = jnp.zeros((128, 128), jnp.int32)


@jax.jit
def kernel(bbox, conf, cls_logits):
    nB, nA, nH, nW, _ = bbox.shape
    g = nB * nA
    bbox_t = bbox.transpose(0, 1, 4, 2, 3).reshape(g, 5, nH, nW)
    conf_s = conf.reshape(g, nH, nW)
    cls_s = cls_logits.reshape(g, nH, nW)

    bbox_o, idx_o, score_o = pl.pallas_call(
        _decode_body,
        grid=(g,),
        in_specs=[
            pl.BlockSpec((1, 5, nH, nW), lambda i: (i, 0, 0, 0)),
            pl.BlockSpec((1, nH, nW), lambda i: (i, 0, 0)),
            pl.BlockSpec((1, nH, nW), lambda i: (i, 0, 0)),
        ],
        out_specs=[
            pl.BlockSpec((5, 1, nH, nW), lambda i: (0, i, 0, 0)),
            pl.BlockSpec((1, nH, nW), lambda i: (i, 0, 0)),
            pl.BlockSpec((1, nH, nW), lambda i: (i, 0, 0)),
        ],
        out_shape=[
            jax.ShapeDtypeStruct((5, g, nH, nW), jnp.float32),
            jax.ShapeDtypeStruct((g, nH, nW), jnp.int32),
            jax.ShapeDtypeStruct((g, nH, nW), jnp.float32),
        ],
    )(bbox_t, conf_s, cls_s)

    n = nA * nH * nW
    bbox_out = bbox_o.reshape(5, nB, n).transpose(1, 2, 0)
    return (bbox_out, idx_o.reshape(nB, n), score_o.reshape(nB, n))
